# Initial kernel scaffold; baseline (speedup 1.0000x reference)
import jax
import jax.numpy as jnp
from jax.experimental import pallas as pl


def kernel(x, table, ch_w, ln_gamma, ln_beta, pos_emb):
    raise NotImplementedError("write your pallas kernel here")


# speedup vs baseline: 1.1421x; 1.1421x over previous
import functools

import jax
import jax.numpy as jnp
from jax import lax
from jax.experimental import pallas as pl
from jax.experimental.pallas import tpu as pltpu
from jax.experimental.pallas import tpu_sc as plsc

VOCAB = 1000000
HIDDEN = 64
NUM_CHANNEL = 4
STEPS = 200
BATCH = 1024

P = BATCH * STEPS
CHUNK = 128
NCHUNK = P // CHUNK
NW = 32
CHUNKS_PER_W = NCHUNK // NW
L = 16
HV = HIDDEN // L


def _rsqrt(v16):
    bits = lax.bitcast_convert_type(v16, jnp.int32)
    y = lax.bitcast_convert_type(
        jnp.int32(0x5F3759DF) - lax.shift_right_logical(bits, 1), jnp.float32)
    for _ in range(3):
        y = y * (1.5 - 0.5 * v16 * y * y)
    return y


def _body(idx_hbm, chw_hbm, gamma_hbm, beta_hbm, pos_hbm, table_hbm,
          out_hbm, idx_v, rows_v, out_v, pos_v, gb_v, w_v, sem):
    wid = lax.axis_index("s") * 2 + lax.axis_index("c")

    pltpu.sync_copy(pos_hbm, pos_v)
    pltpu.sync_copy(gamma_hbm, gb_v.at[0])
    pltpu.sync_copy(beta_hbm, gb_v.at[1])
    pltpu.sync_copy(chw_hbm, w_v)

    e = jnp.exp(w_v[...])
    w = e / jnp.sum(e)
    w0 = w[0]
    w1 = w[1]
    w2 = w[2]
    w3 = w[3]
    gam = [gb_v[0, pl.ds(k * L, L)] for k in range(HV)]
    bet = [gb_v[1, pl.ds(k * L, L)] for k in range(HV)]

    @pl.loop(0, CHUNKS_PER_W)
    def chunk_loop(ci):
        g = wid * CHUNKS_PER_W + ci
        pltpu.sync_copy(idx_hbm.at[g], idx_v)
        cps = [pltpu.async_copy(table_hbm.at[idx_v.at[c]], rows_v.at[c], sem)
               for c in range(NUM_CHANNEL)]
        for cp in cps:
            cp.wait()

        @pl.loop(0, CHUNK)
        def pos_loop(i):
            acc = [w0 * rows_v[0, i, pl.ds(k * L, L)]
                   + w1 * rows_v[1, i, pl.ds(k * L, L)]
                   + w2 * rows_v[2, i, pl.ds(k * L, L)]
                   + w3 * rows_v[3, i, pl.ds(k * L, L)]
                   for k in range(HV)]
            tot = (acc[0] + acc[1]) + (acc[2] + acc[3])
            sq = (acc[0] * acc[0] + acc[1] * acc[1]) + \
                 (acc[2] * acc[2] + acc[3] * acc[3])
            mean = jnp.sum(tot) * (1.0 / HIDDEN)
            var = jnp.sum(sq) * (1.0 / HIDDEN) - mean * mean
            rstd = _rsqrt(jnp.full((L,), var + 1e-5, jnp.float32))
            s = lax.rem(g * CHUNK + i, STEPS)
            for k in range(HV):
                out_v[i, pl.ds(k * L, L)] = (
                    (acc[k] - mean) * rstd * gam[k] + bet[k]
                    + pos_v[s, pl.ds(k * L, L)])

        pltpu.sync_copy(out_v, out_hbm.at[g])


@jax.jit
def kernel(x, table, ch_w, ln_gamma, ln_beta, pos_emb):
    idx = (x.transpose(1, 0, 2).reshape(NUM_CHANNEL, NCHUNK, CHUNK)
           .transpose(1, 0, 2))
    chw16 = jnp.full((L,), -1e30, jnp.float32).at[:NUM_CHANNEL].set(ch_w)

    mesh = plsc.VectorSubcoreMesh(core_axis_name="c", subcore_axis_name="s")
    run = pl.kernel(
        _body,
        out_type=jax.ShapeDtypeStruct((NCHUNK, CHUNK, HIDDEN), jnp.float32),
        mesh=mesh,
        scratch_types=[
            pltpu.VMEM((NUM_CHANNEL, CHUNK), jnp.int32),
            pltpu.VMEM((NUM_CHANNEL, CHUNK, HIDDEN), jnp.float32),
            pltpu.VMEM((CHUNK, HIDDEN), jnp.float32),
            pltpu.VMEM((STEPS, HIDDEN), jnp.float32),
            pltpu.VMEM((2, HIDDEN), jnp.float32),
            pltpu.VMEM((L,), jnp.float32),
            pltpu.SemaphoreType.DMA,
        ],
        compiler_params=pltpu.CompilerParams(
            needs_layout_passes=False, use_tc_tiling_on_sc=False),
    )
    out = run(idx, chw16, ln_gamma, ln_beta, pos_emb, table)
    return out.reshape(BATCH, STEPS, HIDDEN)
